# baseline (speedup 1.0000x reference)
import functools

import jax
import jax.numpy as jnp
from jax import lax
from jax.experimental import pallas as pl
from jax.experimental.pallas import tpu as pltpu
from jax.experimental.pallas import tpu_sc as plsc

HIDDEN = 768
EPS = 1e-12
L = 16
NJ = HIDDEN // L
NC, NS = 2, 16
NW = NC * NS


_GATHER_DNUMS = lax.GatherDimensionNumbers(
    offset_dims=(), collapsed_slice_dims=(0,), start_index_map=(0,))


def _shuffle(x, p):
    return lax.gather(x, p, _GATHER_DNUMS, (1,),
                      mode=lax.GatherScatterMode.PROMISE_IN_BOUNDS)


def _butterfly_sum(x, perms):
    for p in perms:
        x = x + _shuffle(x, p)
    return x


def _rsqrt_vec(x):
    i = lax.bitcast_convert_type(x, jnp.int32)
    i = jnp.full((L,), 0x5F3759DF, jnp.int32) - (i >> 1)
    y = lax.bitcast_convert_type(i, jnp.float32)
    half = x * 0.5
    for _ in range(3):
        y = y * (1.5 - half * y * y)
    return y


def _make_emb_kernel(B, S):
    PPW = S // NW
    mesh = plsc.VectorSubcoreMesh(core_axis_name="c", subcore_axis_name="s")

    @functools.partial(
        pl.kernel,
        out_type=jax.ShapeDtypeStruct((B, S, HIDDEN), jnp.float32),
        mesh=mesh,
        scratch_types=[
            pltpu.VMEM((B * PPW,), jnp.int32),
            pltpu.VMEM((PPW, HIDDEN), jnp.float32),
            pltpu.VMEM((HIDDEN,), jnp.float32),
            pltpu.VMEM((HIDDEN,), jnp.float32),
            pltpu.VMEM((HIDDEN,), jnp.float32),
            pltpu.VMEM((PPW, HIDDEN), jnp.float32),
            pltpu.SemaphoreType.DMA,
        ],
    )
    def emb_kernel(ids, word, pos, typ, gamma, beta, out,
                   idx_v, pos_v, typ_v, gam_v, bet_v, rows_v, sem):
        wid = lax.axis_index("s") * NC + lax.axis_index("c")
        p0 = wid * PPW
        pltpu.sync_copy(pos.at[pl.ds(p0, PPW)], pos_v)
        pltpu.sync_copy(typ.at[0], typ_v)
        pltpu.sync_copy(gamma, gam_v)
        pltpu.sync_copy(beta, bet_v)
        pltpu.sync_copy(ids.at[pl.ds(wid * B * PPW, B * PPW)], idx_v)

        def fold_type(r, c):
            for j in range(NJ):
                sl = pl.ds(j * L, L)
                pos_v[r, sl] = pos_v[r, sl] + typ_v[sl]
            return c
        lax.fori_loop(0, PPW, fold_type, 0)

        lanes = lax.iota(jnp.int32, L)
        perms = [(lanes ^ sh)[:, None] for sh in (8, 4, 2, 1)]

        def batch_body(b, c):
            vidx = idx_v[pl.ds(b * PPW, PPW)]
            pltpu.async_copy(word.at[vidx], rows_v, sem).wait()

            def tok_body(t, tc):
                acc_s = jnp.zeros((L,), jnp.float32)
                acc_q = jnp.zeros((L,), jnp.float32)
                for j in range(NJ):
                    sl = pl.ds(j * L, L)
                    x = rows_v[t, sl] + pos_v[t, sl]
                    rows_v[t, sl] = x
                    acc_s = acc_s + x
                    acc_q = acc_q + x * x
                mean = _butterfly_sum(acc_s, perms) * (1.0 / HIDDEN)
                q = _butterfly_sum(acc_q, perms) * (1.0 / HIDDEN)
                var = q - mean * mean
                rvec = _rsqrt_vec(var + EPS)
                bsc = -mean * rvec
                for j in range(NJ):
                    sl = pl.ds(j * L, L)
                    y = (rows_v[t, sl] * rvec + bsc) * gam_v[sl] + bet_v[sl]
                    rows_v[t, sl] = y
                return tc
            lax.fori_loop(0, PPW, tok_body, 0)
            pltpu.sync_copy(rows_v, out.at[b, pl.ds(p0, PPW)])
            return c
        lax.fori_loop(0, B, batch_body, 0)

    return emb_kernel


def kernel(input_ids, attention_mask, labels, word_emb, pos_emb, type_emb,
           ln_gamma, ln_beta):
    B, S = input_ids.shape
    PPW = S // NW
    ids_stripe = (input_ids.reshape(B, NW, PPW)
                  .transpose(1, 0, 2).reshape(-1))
    out = _make_emb_kernel(B, S)(ids_stripe, word_emb, pos_emb, type_emb,
                                 ln_gamma, ln_beta)
    return (out, labels)


# speedup vs baseline: 1.2625x; 1.2625x over previous
import functools

import jax
import jax.numpy as jnp
from jax import lax
from jax.experimental import pallas as pl
from jax.experimental.pallas import tpu as pltpu
from jax.experimental.pallas import tpu_sc as plsc

HIDDEN = 768
EPS = 1e-12
L = 16
NJ = HIDDEN // L
NC, NS = 2, 16
NW = NC * NS
NBUF = 4

_GATHER_DNUMS = lax.GatherDimensionNumbers(
    offset_dims=(), collapsed_slice_dims=(0,), start_index_map=(0,))


def _shuffle(x, p):
    return lax.gather(x, p, _GATHER_DNUMS, (1,),
                      mode=lax.GatherScatterMode.PROMISE_IN_BOUNDS)


def _butterfly_sum(x, perms):
    for p in perms:
        x = x + _shuffle(x, p)
    return x


def _rsqrt_vec(x):
    i = lax.bitcast_convert_type(x, jnp.int32)
    i = jnp.full((L,), 0x5F3759DF, jnp.int32) - (i >> 1)
    y = lax.bitcast_convert_type(i, jnp.float32)
    half = x * 0.5
    for _ in range(3):
        y = y * (1.5 - half * y * y)
    return y


def _make_emb_kernel(B, S):
    PPW = S // NW
    ROW_BYTES = PPW * HIDDEN * 4
    mesh = plsc.VectorSubcoreMesh(core_axis_name="c", subcore_axis_name="s")

    @functools.partial(
        pl.kernel,
        out_type=jax.ShapeDtypeStruct((B, S, HIDDEN), jnp.float32),
        mesh=mesh,
        scratch_types=[
            pltpu.VMEM((B * PPW,), jnp.int32),
            pltpu.VMEM((PPW, HIDDEN), jnp.float32),
            pltpu.VMEM((HIDDEN,), jnp.float32),
            pltpu.VMEM((HIDDEN,), jnp.float32),
            pltpu.VMEM((HIDDEN,), jnp.float32),
            [pltpu.VMEM((PPW, HIDDEN), jnp.float32) for _ in range(NBUF)],
            [pltpu.SemaphoreType.DMA for _ in range(NBUF)],
            [pltpu.SemaphoreType.DMA for _ in range(NBUF)],
        ],
    )
    def emb_kernel(ids, word, pos, typ, gamma, beta, out,
                   idx_v, pos_v, typ_v, gam_v, bet_v, bufs, gsems, ssems):
        wid = lax.axis_index("s") * NC + lax.axis_index("c")
        p0 = wid * PPW
        pltpu.sync_copy(pos.at[pl.ds(p0, PPW)], pos_v)
        pltpu.sync_copy(typ.at[0], typ_v)
        pltpu.sync_copy(gamma, gam_v)
        pltpu.sync_copy(beta, bet_v)
        pltpu.sync_copy(ids.at[pl.ds(wid * B * PPW, B * PPW)], idx_v)

        def fold_type(r, c):
            for j in range(NJ):
                sl = pl.ds(j * L, L)
                pos_v[r, sl] = pos_v[r, sl] + typ_v[sl]
            return c
        lax.fori_loop(0, PPW, fold_type, 0)

        lanes = lax.iota(jnp.int32, L)
        perms = [(lanes ^ sh)[:, None] for sh in (8, 4, 2, 1)]

        def fire_gather(b, k):
            vidx = idx_v[pl.ds(b * PPW, PPW)]
            pltpu.async_copy(word.at[vidx], bufs[k], gsems[k])

        def wait_gather(k):
            pltpu.make_async_copy(word.at[pl.ds(0, PPW)], bufs[k],
                                  gsems[k]).wait()

        def wait_store(k):
            pltpu.make_async_copy(bufs[k], out.at[0, pl.ds(p0, PPW)],
                                  ssems[k]).wait()

        def compute(rows_v):
            def tok_body(t, tc):
                a0 = jnp.zeros((L,), jnp.float32)
                a1 = jnp.zeros((L,), jnp.float32)
                a2 = jnp.zeros((L,), jnp.float32)
                a3 = jnp.zeros((L,), jnp.float32)
                q0 = jnp.zeros((L,), jnp.float32)
                q1 = jnp.zeros((L,), jnp.float32)
                q2 = jnp.zeros((L,), jnp.float32)
                q3 = jnp.zeros((L,), jnp.float32)
                for j in range(0, NJ, 4):
                    for u in range(4):
                        sl = pl.ds((j + u) * L, L)
                        x = rows_v[t, sl] + pos_v[t, sl]
                        rows_v[t, sl] = x
                        if u == 0:
                            a0 = a0 + x
                            q0 = q0 + x * x
                        elif u == 1:
                            a1 = a1 + x
                            q1 = q1 + x * x
                        elif u == 2:
                            a2 = a2 + x
                            q2 = q2 + x * x
                        else:
                            a3 = a3 + x
                            q3 = q3 + x * x
                acc_s = (a0 + a1) + (a2 + a3)
                acc_q = (q0 + q1) + (q2 + q3)
                mean = _butterfly_sum(acc_s, perms) * (1.0 / HIDDEN)
                q = _butterfly_sum(acc_q, perms) * (1.0 / HIDDEN)
                var = q - mean * mean
                rvec = _rsqrt_vec(var + EPS)
                bsc = -mean * rvec
                for j in range(NJ):
                    sl = pl.ds(j * L, L)
                    y = (rows_v[t, sl] * rvec + bsc) * gam_v[sl] + bet_v[sl]
                    rows_v[t, sl] = y
                return tc
            lax.fori_loop(0, PPW, tok_body, 0)

        fire_gather(0, 0)
        fire_gather(1, 1)

        def group_body(g, c):
            for k in range(NBUF):
                b = g * NBUF + k
                wait_gather(k)
                compute(bufs[k])
                pltpu.async_copy(bufs[k], out.at[b, pl.ds(p0, PPW)], ssems[k])
                if k >= 2:
                    wait_store((k + 2) % NBUF)
                else:
                    @pl.when(g >= 1)
                    def _():
                        wait_store((k + 2) % NBUF)

                @pl.when(b + 2 < B)
                def _():
                    fire_gather(b + 2, (k + 2) % NBUF)
            return c
        lax.fori_loop(0, B // NBUF, group_body, 0)

        wait_store(2)
        wait_store(3)

    return emb_kernel


def kernel(input_ids, attention_mask, labels, word_emb, pos_emb, type_emb,
           ln_gamma, ln_beta):
    B, S = input_ids.shape
    PPW = S // NW
    ids_stripe = (input_ids.reshape(B, NW, PPW)
                  .transpose(1, 0, 2).reshape(-1))
    out = _make_emb_kernel(B, S)(ids_stripe, word_emb, pos_emb, type_emb,
                                 ln_gamma, ln_beta)
    return (out, labels)
